# Initial kernel scaffold; baseline (speedup 1.0000x reference)
import jax
import jax.numpy as jnp
from jax.experimental import pallas as pl


def kernel(x_QT, x_SLI, eta, W_QT, b_QT, W_SLI, b_SLI):
    raise NotImplementedError("write your pallas kernel here")


# speedup vs baseline: 1.8883x; 1.8883x over previous
import jax
import jax.numpy as jnp
from jax.experimental import pallas as pl

NZ = 34
E = 7
TILE = 2048


def _moe_kernel(eta_ref, xq_ref, xs_ref, wq_ref, bq_ref, ws_ref, bs_ref, out_ref):
    xq = xq_ref[...].astype(jnp.bfloat16)
    xs = xs_ref[...].astype(jnp.bfloat16)
    pq = jnp.dot(wq_ref[...], xq, preferred_element_type=jnp.float32)
    ps = jnp.dot(ws_ref[...], xs, preferred_element_type=jnp.float32)
    pq = pq + bq_ref[...]
    ps = ps + bs_ref[...]
    eta = eta_ref[0]
    accq = jnp.zeros((NZ, TILE), jnp.float32)
    accs = jnp.zeros((NZ, TILE), jnp.float32)
    for e in range(E):
        m = eta == e
        accq = jnp.where(m, pq[e * NZ:(e + 1) * NZ, :], accq)
        accs = jnp.where(m, ps[e * NZ:(e + 1) * NZ, :], accs)
    out_ref[0] = accq
    out_ref[1] = accs


def kernel(x_QT, x_SLI, eta, W_QT, b_QT, W_SLI, b_SLI):
    NY, NX = eta.shape
    NC = NY * NX
    G = NC // TILE
    xq = x_QT.reshape(NZ, NC)
    xs = x_SLI.reshape(NZ, NC)
    eta3 = eta.reshape(G, 1, TILE)
    wq = W_QT.reshape(E * NZ, NZ).astype(jnp.bfloat16)
    ws = W_SLI.reshape(E * NZ, NZ).astype(jnp.bfloat16)
    bq = b_QT.reshape(E * NZ, 1)
    bs = b_SLI.reshape(E * NZ, 1)
    out = pl.pallas_call(
        _moe_kernel,
        grid=(G,),
        in_specs=[
            pl.BlockSpec((1, 1, TILE), lambda i: (i, 0, 0)),
            pl.BlockSpec((NZ, TILE), lambda i: (0, i)),
            pl.BlockSpec((NZ, TILE), lambda i: (0, i)),
            pl.BlockSpec((E * NZ, NZ), lambda i: (0, 0)),
            pl.BlockSpec((E * NZ, 1), lambda i: (0, 0)),
            pl.BlockSpec((E * NZ, NZ), lambda i: (0, 0)),
            pl.BlockSpec((E * NZ, 1), lambda i: (0, 0)),
        ],
        out_specs=pl.BlockSpec((2, NZ, TILE), lambda i: (0, 0, i)),
        out_shape=jax.ShapeDtypeStruct((2, NZ, NC), jnp.float32),
    )(eta3, xq, xs, wq, bq, ws, bs)
    return out.reshape(2, NZ, NY, NX)


# speedup vs baseline: 5.5058x; 2.9157x over previous
import jax
import jax.numpy as jnp
from jax.experimental import pallas as pl

NZ = 34
E = 7
S = 40
KX = E * S
R = 8


def _moe_kernel(eta_ref, xq_ref, xs_ref, wq_ref, ws_ref, out_ref):
    _, ny, nx = xq_ref.shape
    n = ny * nx
    eta = eta_ref[0]
    xq = xq_ref[...].reshape(NZ, n).astype(jnp.bfloat16)
    xs = xs_ref[...].reshape(NZ, n).astype(jnp.bfloat16)
    pad = jnp.zeros((S - NZ - 1, n), jnp.bfloat16)
    one = jnp.ones((1, n), jnp.bfloat16)
    xaugq = jnp.concatenate([xq, one, pad], axis=0)
    xaugs = jnp.concatenate([xs, one, pad], axis=0)
    zed = jnp.zeros((S, n), jnp.bfloat16)
    xkq = jnp.concatenate([jnp.where(eta == e, xaugq, zed) for e in range(E)], axis=0)
    xks = jnp.concatenate([jnp.where(eta == e, xaugs, zed) for e in range(E)], axis=0)
    oq = jnp.dot(wq_ref[...], xkq, preferred_element_type=jnp.float32)
    osli = jnp.dot(ws_ref[...], xks, preferred_element_type=jnp.float32)
    out_ref[0] = oq.reshape(NZ, ny, nx)
    out_ref[1] = osli.reshape(NZ, ny, nx)


def _stack_weights(W, b):
    wc = jnp.zeros((NZ, E, S), W.dtype)
    wc = wc.at[:, :, :NZ].set(jnp.transpose(W, (1, 0, 2)))
    wc = wc.at[:, :, NZ].set(b.T)
    return wc.reshape(NZ, KX).astype(jnp.bfloat16)


def kernel(x_QT, x_SLI, eta, W_QT, b_QT, W_SLI, b_SLI):
    NY, NX = eta.shape
    G = NY // R
    wq = _stack_weights(W_QT, b_QT)
    ws = _stack_weights(W_SLI, b_SLI)
    eta3 = eta.reshape(G, 1, R * NX)
    return pl.pallas_call(
        _moe_kernel,
        grid=(G,),
        in_specs=[
            pl.BlockSpec((1, 1, R * NX), lambda i: (i, 0, 0)),
            pl.BlockSpec((NZ, R, NX), lambda i: (0, i, 0)),
            pl.BlockSpec((NZ, R, NX), lambda i: (0, i, 0)),
            pl.BlockSpec((NZ, KX), lambda i: (0, 0)),
            pl.BlockSpec((NZ, KX), lambda i: (0, 0)),
        ],
        out_specs=pl.BlockSpec((2, NZ, R, NX), lambda i: (0, 0, i, 0)),
        out_shape=jax.ShapeDtypeStruct((2, NZ, NY, NX), jnp.float32),
    )(eta3, x_QT, x_SLI, wq, ws)


# speedup vs baseline: 6.1027x; 1.1084x over previous
import jax
import jax.numpy as jnp
from jax.experimental import pallas as pl

NZ = 34
E = 7
S = 40
KX = E * S
R = 16


def _moe_kernel(eta_ref, xq_ref, xs_ref, wq_ref, ws_ref, out_ref):
    _, ny, nx = xq_ref.shape
    n = ny * nx
    eta = eta_ref[0]
    xq = xq_ref[...].astype(jnp.bfloat16).reshape(NZ, n)
    xs = xs_ref[...].astype(jnp.bfloat16).reshape(NZ, n)
    pad = jnp.zeros((S - NZ - 1, n), jnp.bfloat16)
    one = jnp.ones((1, n), jnp.bfloat16)
    xaugq = jnp.concatenate([xq, one, pad], axis=0)
    xaugs = jnp.concatenate([xs, one, pad], axis=0)
    zed = jnp.zeros((S, n), jnp.bfloat16)
    xkq = jnp.concatenate([jnp.where(eta == e, xaugq, zed) for e in range(E)], axis=0)
    xks = jnp.concatenate([jnp.where(eta == e, xaugs, zed) for e in range(E)], axis=0)
    oq = jnp.dot(wq_ref[...], xkq, preferred_element_type=jnp.float32)
    osli = jnp.dot(ws_ref[...], xks, preferred_element_type=jnp.float32)
    out_ref[0] = oq.reshape(NZ, ny, nx)
    out_ref[1] = osli.reshape(NZ, ny, nx)


def _stack_weights(W, b):
    wc = jnp.zeros((NZ, E, S), W.dtype)
    wc = wc.at[:, :, :NZ].set(jnp.transpose(W, (1, 0, 2)))
    wc = wc.at[:, :, NZ].set(b.T)
    return wc.reshape(NZ, KX).astype(jnp.bfloat16)


def kernel(x_QT, x_SLI, eta, W_QT, b_QT, W_SLI, b_SLI):
    NY, NX = eta.shape
    G = NY // R
    wq = _stack_weights(W_QT, b_QT)
    ws = _stack_weights(W_SLI, b_SLI)
    eta3 = eta.reshape(G, 1, R * NX)
    return pl.pallas_call(
        _moe_kernel,
        grid=(G,),
        in_specs=[
            pl.BlockSpec((1, 1, R * NX), lambda i: (i, 0, 0)),
            pl.BlockSpec((NZ, R, NX), lambda i: (0, i, 0)),
            pl.BlockSpec((NZ, R, NX), lambda i: (0, i, 0)),
            pl.BlockSpec((NZ, KX), lambda i: (0, 0)),
            pl.BlockSpec((NZ, KX), lambda i: (0, 0)),
        ],
        out_specs=pl.BlockSpec((2, NZ, R, NX), lambda i: (0, 0, i, 0)),
        out_shape=jax.ShapeDtypeStruct((2, NZ, NY, NX), jnp.float32),
    )(eta3, x_QT, x_SLI, wq, ws)
